# speedup vs baseline: 1.5428x; 1.5428x over previous
import functools
import jax
import jax.numpy as jnp
from jax import lax
from jax.experimental import pallas as pl
from jax.experimental.pallas import tpu as pltpu
from jax.experimental.pallas import tpu_sc as plsc

VOCAB = 151936
DIM = 1152
B = 8
S = 2048
NTOK = B * S


@functools.lru_cache(maxsize=None)
def _build_gather():
    info = plsc.get_sparse_core_info()
    nc, ns = info.num_cores, info.num_subcores
    nw = nc * ns
    rows_per_w = NTOK // nw
    chunk = 64
    nchunk = rows_per_w // chunk

    mesh = plsc.VectorSubcoreMesh(core_axis_name="c", subcore_axis_name="s")

    @functools.partial(
        pl.kernel,
        mesh=mesh,
        out_type=jax.ShapeDtypeStruct((NTOK, DIM), jnp.float32),
        scratch_types=[
            pltpu.VMEM((chunk,), jnp.int32),
            pltpu.VMEM((chunk, DIM), jnp.float32),
            pltpu.SemaphoreType.DMA,
        ],
    )
    def gather(idx_hbm, table_hbm, out_hbm, idx_v, rows_v, sem):
        wid = lax.axis_index("s") * nc + lax.axis_index("c")
        base = wid * rows_per_w
        for g in range(nchunk):
            off = base + g * chunk
            pltpu.sync_copy(idx_hbm.at[pl.ds(off, chunk)], idx_v)
            pltpu.async_copy(table_hbm.at[idx_v], rows_v, sem).wait()
            pltpu.sync_copy(rows_v, out_hbm.at[pl.ds(off, chunk)])

    return gather


def kernel(x, emb_weight):
    idx = x.reshape(NTOK).astype(jnp.int32)
    out = _build_gather()(idx, emb_weight)
    return out.reshape(B, S, DIM)


# speedup vs baseline: 1.6644x; 1.0788x over previous
import functools
import jax
import jax.numpy as jnp
from jax import lax
from jax.experimental import pallas as pl
from jax.experimental.pallas import tpu as pltpu
from jax.experimental.pallas import tpu_sc as plsc

VOCAB = 151936
DIM = 1152
B = 8
S = 2048
NTOK = B * S


@functools.lru_cache(maxsize=None)
def _build_gather():
    info = plsc.get_sparse_core_info()
    nc, ns = info.num_cores, info.num_subcores
    nw = nc * ns
    rows_per_w = NTOK // nw
    chunk = 32
    nbuf = 3
    nchunk = rows_per_w // chunk

    mesh = plsc.VectorSubcoreMesh(core_axis_name="c", subcore_axis_name="s")

    @functools.partial(
        pl.kernel,
        mesh=mesh,
        out_type=jax.ShapeDtypeStruct((NTOK, DIM), jnp.float32),
        scratch_types=[
            pltpu.VMEM((rows_per_w,), jnp.int32),
        ]
        + [pltpu.VMEM((chunk, DIM), jnp.float32) for _ in range(nbuf)]
        + [pltpu.SemaphoreType.DMA for _ in range(2 * nbuf)],
    )
    def gather(idx_hbm, table_hbm, out_hbm, idx_v, *bufs_and_sems):
        bufs = bufs_and_sems[:nbuf]
        sem_g = bufs_and_sems[nbuf:2 * nbuf]
        sem_o = bufs_and_sems[2 * nbuf:]
        wid = lax.axis_index("s") * nc + lax.axis_index("c")
        base = wid * rows_per_w
        pltpu.sync_copy(idx_hbm.at[pl.ds(base, rows_per_w)], idx_v)

        def start_gather(g):
            p = g % nbuf
            pltpu.async_copy(
                table_hbm.at[idx_v.at[pl.ds(g * chunk, chunk)]],
                bufs[p], sem_g[p])

        def start_out(g):
            p = g % nbuf
            pltpu.async_copy(
                bufs[p], out_hbm.at[pl.ds(base + g * chunk, chunk)],
                sem_o[p])

        start_gather(0)
        start_gather(1)
        outs_waited = set()
        for g in range(nchunk):
            if g + 2 < nchunk:
                if g - 1 >= 0:
                    pltpu.make_async_copy(
                        bufs[(g - 1) % nbuf],
                        out_hbm.at[pl.ds(base + (g - 1) * chunk, chunk)],
                        sem_o[(g - 1) % nbuf]).wait()
                    outs_waited.add(g - 1)
                start_gather(g + 2)
            pltpu.make_async_copy(
                table_hbm.at[idx_v.at[pl.ds(g * chunk, chunk)]],
                bufs[g % nbuf], sem_g[g % nbuf]).wait()
            start_out(g)
        for g in range(nchunk):
            if g not in outs_waited:
                pltpu.make_async_copy(
                    bufs[g % nbuf],
                    out_hbm.at[pl.ds(base + g * chunk, chunk)],
                    sem_o[g % nbuf]).wait()

    return gather


def kernel(x, emb_weight):
    idx = x.reshape(NTOK).astype(jnp.int32)
    out = _build_gather()(idx, emb_weight)
    return out.reshape(B, S, DIM)
